# baseline (speedup 1.0000x reference)
import jax
import jax.numpy as jnp
from jax.experimental import pallas as pl
from jax.experimental.pallas import tpu as pltpu


def _copy_body(lens_ref, in_ref, out_ref):
    out_ref[...] = in_ref[...]


def kernel(input_, lengths):
    B, T, D = input_.shape
    lens = lengths.astype(jnp.int32)

    flat = input_.reshape(B * T, 1, D)

    def in_map(b, lens_ref):
        n = lens_ref[b]
        row = jnp.where(n > 0, n - 1, T - 1)
        return (b * T + row, 0, 0)

    grid_spec = pltpu.PrefetchScalarGridSpec(
        num_scalar_prefetch=1,
        grid=(B,),
        in_specs=[pl.BlockSpec((1, 1, D), in_map)],
        out_specs=pl.BlockSpec((1, 1, D), lambda b, lens_ref: (b, 0, 0)),
    )
    return pl.pallas_call(
        _copy_body,
        grid_spec=grid_spec,
        out_shape=jax.ShapeDtypeStruct((B, 1, D), input_.dtype),
    )(lens, flat)


# speedup vs baseline: 154.8426x; 154.8426x over previous
import jax
import jax.numpy as jnp
from jax.experimental import pallas as pl
from jax.experimental.pallas import tpu as pltpu


def _gather_body(lens_ref, in_hbm, out_hbm, sem):
    B, T, _ = in_hbm.shape
    copies = []
    for b in range(B):
        n = lens_ref[b]
        row = jnp.where(n > 0, n - 1, T - 1)
        cp = pltpu.make_async_copy(in_hbm.at[b, row], out_hbm.at[b, 0], sem)
        cp.start()
        copies.append(cp)
    for cp in copies:
        cp.wait()


def kernel(input_, lengths):
    B, T, D = input_.shape
    lens = lengths.astype(jnp.int32)

    grid_spec = pltpu.PrefetchScalarGridSpec(
        num_scalar_prefetch=1,
        grid=(1,),
        in_specs=[pl.BlockSpec(memory_space=pltpu.MemorySpace.HBM)],
        out_specs=pl.BlockSpec(memory_space=pltpu.MemorySpace.HBM),
        scratch_shapes=[pltpu.SemaphoreType.DMA],
    )
    return pl.pallas_call(
        _gather_body,
        grid_spec=grid_spec,
        out_shape=jax.ShapeDtypeStruct((B, 1, D), input_.dtype),
    )(lens, input_)


# speedup vs baseline: 156.7162x; 1.0121x over previous
import jax
import jax.numpy as jnp
from jax.experimental import pallas as pl
from jax.experimental.pallas import tpu as pltpu


def _gather_body(lens_ref, in_hbm, out_hbm, sem):
    B, T, _ = in_hbm.shape
    copies = []
    for b in range(B):
        n = lens_ref[b]
        row = jnp.where(n > 0, n - 1, T - 1)
        cp = pltpu.make_async_copy(in_hbm.at[b, row], out_hbm.at[b, 0], sem)
        cp.start()
        copies.append(cp)
    for cp in copies:
        cp.wait()


def kernel(input_, lengths):
    B, T, D = input_.shape
    lens = lengths.astype(jnp.int32)

    return pl.pallas_call(
        _gather_body,
        in_specs=[
            pl.BlockSpec(memory_space=pltpu.MemorySpace.SMEM),
            pl.BlockSpec(memory_space=pltpu.MemorySpace.HBM),
        ],
        out_specs=pl.BlockSpec(memory_space=pltpu.MemorySpace.HBM),
        scratch_shapes=[pltpu.SemaphoreType.DMA],
        out_shape=jax.ShapeDtypeStruct((B, 1, D), input_.dtype),
    )(lens, input_)


# speedup vs baseline: 203.5170x; 1.2986x over previous
import jax
import jax.numpy as jnp
from jax.experimental import pallas as pl
from jax.experimental.pallas import tpu as pltpu


def _gather_body(lens_ref, in_hbm, out_ref, sem):
    B, T, _ = in_hbm.shape
    copies = []
    for b in range(B):
        n = lens_ref[b]
        row = jnp.where(n > 0, n - 1, T - 1)
        cp = pltpu.make_async_copy(in_hbm.at[b, row], out_ref.at[b, 0], sem)
        cp.start()
        copies.append(cp)
    for cp in copies:
        cp.wait()


def kernel(input_, lengths):
    B, T, D = input_.shape
    lens = lengths.astype(jnp.int32)

    return pl.pallas_call(
        _gather_body,
        in_specs=[
            pl.BlockSpec(memory_space=pltpu.MemorySpace.SMEM),
            pl.BlockSpec(memory_space=pltpu.MemorySpace.HBM),
        ],
        out_specs=pl.BlockSpec(memory_space=pltpu.MemorySpace.VMEM),
        scratch_shapes=[pltpu.SemaphoreType.DMA],
        out_shape=jax.ShapeDtypeStruct((B, 1, D), input_.dtype),
    )(lens, input_)
